# baseline (speedup 1.0000x reference)
import functools

import jax
import jax.numpy as jnp
from jax import lax
from jax.experimental import pallas as pl
from jax.experimental.pallas import tpu as pltpu
from jax.experimental.pallas import tpu_sc as plsc

NUM_NODES = 64
NUM_INPUTS = 8
NUM_EDGES = 256
N_HIDDEN = NUM_NODES - NUM_INPUTS
A_LEN = N_HIDDEN * NUM_INPUTS
LANES = 16
TILE = 65536


def _sc_edge_aggregate_body(src_hbm, dst_hbm, w_hbm, a_hbm,
                            src_v, dst_v, w_v, key_lo, key_hi, a_v, shared,
                            sem0, sem1, sem2):
    cid = lax.axis_index("c")
    sid = lax.axis_index("s")

    @pl.when(jnp.logical_and(cid == 0, sid == 0))
    def _():
        cp0 = pltpu.async_copy(src_hbm, src_v, sem0)
        cp1 = pltpu.async_copy(dst_hbm, dst_v, sem1)
        cp2 = pltpu.async_copy(w_hbm, w_v, sem2)

        zeros16 = jnp.zeros((LANES,), jnp.float32)
        for c in range(A_LEN // LANES):
            a_v[pl.ds(c * LANES, LANES)] = zeros16
        pltpu.sync_copy(a_v, shared)

        cp0.wait()
        cp1.wait()
        cp2.wait()

        for c in range(NUM_EDGES // LANES):
            s16 = src_v[pl.ds(c * LANES, LANES)]
            d16 = dst_v[pl.ds(c * LANES, LANES)]
            k16 = (d16 - NUM_INPUTS) * NUM_INPUTS + s16
            if c < 8:
                key_lo[pl.ds(c * LANES, LANES)] = k16
            else:
                key_hi[pl.ds((c - 8) * LANES, LANES)] = k16

        cps0 = pltpu.async_copy(w_v.at[pl.ds(0, 128)], shared.at[key_lo],
                                sem0, add=True)
        cps1 = pltpu.async_copy(w_v.at[pl.ds(128, 128)], shared.at[key_hi],
                                sem1, add=True)
        cps0.wait()
        cps1.wait()
        pltpu.sync_copy(shared, a_hbm)


_sc_edge_aggregate = functools.partial(
    pl.kernel,
    mesh=plsc.VectorSubcoreMesh(core_axis_name="c", subcore_axis_name="s"),
    out_type=jax.ShapeDtypeStruct((A_LEN,), jnp.float32),
    scratch_types=[
        pltpu.VMEM((NUM_EDGES,), jnp.int32),
        pltpu.VMEM((NUM_EDGES,), jnp.int32),
        pltpu.VMEM((NUM_EDGES,), jnp.float32),
        pltpu.VMEM((128,), jnp.int32),
        pltpu.VMEM((128,), jnp.int32),
        pltpu.VMEM((A_LEN,), jnp.float32),
        pltpu.VMEM_SHARED((A_LEN,), jnp.float32),
        pltpu.SemaphoreType.DMA,
        pltpu.SemaphoreType.DMA,
        pltpu.SemaphoreType.DMA,
    ],
)(_sc_edge_aggregate_body)


def _tc_dense_kernel(a_ref, loss_ref, prev_ref, params_ref, out_ref):
    feat = jnp.concatenate([loss_ref[:], prev_ref[:], params_ref[:]], axis=0)
    out_ref[:] = jnp.dot(a_ref[:], feat, preferred_element_type=jnp.float32)


@jax.jit
def kernel(loss, prev_loss, params, weights, edge_src, edge_dst):
    n = loss.shape[0]
    a_flat = _sc_edge_aggregate(edge_src, edge_dst, weights)
    a = a_flat.reshape(N_HIDDEN, NUM_INPUTS)

    grid = (n // TILE,)
    out = pl.pallas_call(
        _tc_dense_kernel,
        grid=grid,
        in_specs=[
            pl.BlockSpec((N_HIDDEN, NUM_INPUTS), lambda i: (0, 0)),
            pl.BlockSpec((1, TILE), lambda i: (0, i)),
            pl.BlockSpec((1, TILE), lambda i: (0, i)),
            pl.BlockSpec((6, TILE), lambda i: (0, i)),
        ],
        out_specs=pl.BlockSpec((N_HIDDEN, TILE), lambda i: (0, i)),
        out_shape=jax.ShapeDtypeStruct((N_HIDDEN, n), jnp.float32),
        compiler_params=pltpu.CompilerParams(
            dimension_semantics=("arbitrary",)),
    )(a, loss[None, :], prev_loss[None, :], params)
    return out


# speedup vs baseline: 1.0423x; 1.0423x over previous
import functools

import jax
import jax.numpy as jnp
from jax import lax
from jax.experimental import pallas as pl
from jax.experimental.pallas import tpu as pltpu
from jax.experimental.pallas import tpu_sc as plsc

NUM_NODES = 64
NUM_INPUTS = 8
NUM_EDGES = 256
N_HIDDEN = NUM_NODES - NUM_INPUTS
A_LEN = N_HIDDEN * NUM_INPUTS
LANES = 16
TILE = 65536


def _sc_edge_aggregate_body(src_hbm, dst_hbm, w_hbm, a_hbm,
                            src_v, dst_v, w_v, key_lo, key_hi, a_v, shared,
                            sem0, sem1, sem2):
    cid = lax.axis_index("c")
    sid = lax.axis_index("s")

    @pl.when(jnp.logical_and(cid == 0, sid == 0))
    def _():
        cp0 = pltpu.async_copy(src_hbm, src_v, sem0)
        cp1 = pltpu.async_copy(dst_hbm, dst_v, sem1)
        cp2 = pltpu.async_copy(w_hbm, w_v, sem2)

        zeros16 = jnp.zeros((LANES,), jnp.float32)
        for c in range(A_LEN // LANES):
            a_v[pl.ds(c * LANES, LANES)] = zeros16
        pltpu.sync_copy(a_v, shared)

        cp0.wait()
        cp1.wait()
        cp2.wait()

        for c in range(NUM_EDGES // LANES):
            s16 = src_v[pl.ds(c * LANES, LANES)]
            d16 = dst_v[pl.ds(c * LANES, LANES)]
            k16 = (d16 - NUM_INPUTS) * NUM_INPUTS + s16
            if c < 8:
                key_lo[pl.ds(c * LANES, LANES)] = k16
            else:
                key_hi[pl.ds((c - 8) * LANES, LANES)] = k16

        cps0 = pltpu.async_copy(w_v.at[pl.ds(0, 128)], shared.at[key_lo],
                                sem0, add=True)
        cps1 = pltpu.async_copy(w_v.at[pl.ds(128, 128)], shared.at[key_hi],
                                sem1, add=True)
        cps0.wait()
        cps1.wait()
        pltpu.sync_copy(shared, a_hbm)


_sc_edge_aggregate = functools.partial(
    pl.kernel,
    mesh=plsc.VectorSubcoreMesh(core_axis_name="c", subcore_axis_name="s",
                                num_cores=1),
    out_type=jax.ShapeDtypeStruct((A_LEN,), jnp.float32),
    scratch_types=[
        pltpu.VMEM((NUM_EDGES,), jnp.int32),
        pltpu.VMEM((NUM_EDGES,), jnp.int32),
        pltpu.VMEM((NUM_EDGES,), jnp.float32),
        pltpu.VMEM((128,), jnp.int32),
        pltpu.VMEM((128,), jnp.int32),
        pltpu.VMEM((A_LEN,), jnp.float32),
        pltpu.VMEM_SHARED((A_LEN,), jnp.float32),
        pltpu.SemaphoreType.DMA,
        pltpu.SemaphoreType.DMA,
        pltpu.SemaphoreType.DMA,
    ],
)(_sc_edge_aggregate_body)


def _tc_dense_kernel(a_ref, loss_ref, prev_ref, params_ref, out_ref):
    feat = jnp.concatenate([loss_ref[:], prev_ref[:], params_ref[:]], axis=0)
    out_ref[:] = jnp.dot(a_ref[:], feat, preferred_element_type=jnp.float32)


@jax.jit
def kernel(loss, prev_loss, params, weights, edge_src, edge_dst):
    n = loss.shape[0]
    a_flat = _sc_edge_aggregate(edge_src, edge_dst, weights)
    a = a_flat.reshape(N_HIDDEN, NUM_INPUTS)

    grid = (n // TILE,)
    out = pl.pallas_call(
        _tc_dense_kernel,
        grid=grid,
        in_specs=[
            pl.BlockSpec((N_HIDDEN, NUM_INPUTS), lambda i: (0, 0)),
            pl.BlockSpec((1, TILE), lambda i: (0, i)),
            pl.BlockSpec((1, TILE), lambda i: (0, i)),
            pl.BlockSpec((6, TILE), lambda i: (0, i)),
        ],
        out_specs=pl.BlockSpec((N_HIDDEN, TILE), lambda i: (0, i)),
        out_shape=jax.ShapeDtypeStruct((N_HIDDEN, n), jnp.float32),
        compiler_params=pltpu.CompilerParams(
            dimension_semantics=("arbitrary",)),
    )(a, loss[None, :], prev_loss[None, :], params)
    return out
